# baseline (speedup 1.0000x reference)
import functools

import jax
import jax.numpy as jnp
from jax.experimental import pallas as pl
from jax.experimental.pallas import tpu as pltpu

_CORES = 2


def _moe_kernel(idx_ref, w_ref, x_ref, scale_ref, gu_ref, dw_ref, out_ref, *,
                F, e_per_core):
    c = pl.program_id(0)
    ei = pl.program_id(1)
    e = c * e_per_core + ei

    @pl.when(ei == 0)
    def _init():
        out_ref[...] = jnp.zeros_like(out_ref)

    mask = idx_ref[...] == e
    coeff = jnp.sum(jnp.where(mask, w_ref[...], 0.0), axis=1, keepdims=True)
    coeff = coeff * scale_ref[e]

    x = x_ref[...].astype(jnp.bfloat16)
    gu = gu_ref[0].astype(jnp.bfloat16)
    h = jnp.dot(x, gu, preferred_element_type=jnp.float32)
    gate = h[:, :F]
    up = h[:, F:]
    act = 0.5 * gate * (1.0 + jax.lax.erf(gate * 0.7071067811865476)) * up
    y = jnp.dot(act.astype(jnp.bfloat16), dw_ref[0].astype(jnp.bfloat16),
                preferred_element_type=jnp.float32)
    out_ref[0] += coeff * y


def kernel(x, weights, expert_indices, gate_up, down, per_expert_scale):
    B, L, D = x.shape
    K = weights.shape[-1]
    E, _, F2 = gate_up.shape
    F = F2 // 2
    N = B * L
    e_per_core = E // _CORES

    x_flat = x.reshape(N, D)
    w_flat = weights.reshape(N, K)
    idx_flat = expert_indices.reshape(N, K)

    partial = pl.pallas_call(
        functools.partial(_moe_kernel, F=F, e_per_core=e_per_core),
        grid=(_CORES, e_per_core),
        in_specs=[
            pl.BlockSpec((N, K), lambda c, e: (0, 0)),
            pl.BlockSpec((N, K), lambda c, e: (0, 0)),
            pl.BlockSpec((N, D), lambda c, e: (0, 0)),
            pl.BlockSpec(memory_space=pltpu.SMEM),
            pl.BlockSpec((1, D, F2), lambda c, e: (c * (E // _CORES) + e, 0, 0)),
            pl.BlockSpec((1, F, D), lambda c, e: (c * (E // _CORES) + e, 0, 0)),
        ],
        out_specs=pl.BlockSpec((1, N, D), lambda c, e: (c, 0, 0)),
        out_shape=jax.ShapeDtypeStruct((_CORES, N, D), jnp.float32),
        compiler_params=pltpu.CompilerParams(
            dimension_semantics=("parallel", "arbitrary"),
        ),
    )(idx_flat, w_flat, x_flat, per_expert_scale, gate_up, down)

    return (partial[0] + partial[1]).reshape(B, L, D)


# speedup vs baseline: 1.0249x; 1.0249x over previous
import functools

import jax
import jax.numpy as jnp
from jax.experimental import pallas as pl
from jax.experimental.pallas import tpu as pltpu

_CORES = 2


def _moe_kernel(idx_ref, w_ref, x_ref, scale_ref, gu_ref, dw_ref, out_ref, *,
                F):
    e = pl.program_id(0)

    @pl.when(e == 0)
    def _init():
        out_ref[...] = jnp.zeros_like(out_ref)

    mask = idx_ref[...] == e
    coeff = jnp.sum(jnp.where(mask, w_ref[...], 0.0), axis=1, keepdims=True)
    coeff = coeff * scale_ref[e]

    x = x_ref[...].astype(jnp.bfloat16)
    gu = gu_ref[0].astype(jnp.bfloat16)
    h = jnp.dot(x, gu, preferred_element_type=jnp.float32)
    gate = h[:, :F]
    up = h[:, F:]
    act = 0.5 * gate * (1.0 + jax.lax.erf(gate * 0.7071067811865476)) * up
    y = jnp.dot(act.astype(jnp.bfloat16), dw_ref[0].astype(jnp.bfloat16),
                preferred_element_type=jnp.float32)
    out_ref[...] += coeff * y


def kernel(x, weights, expert_indices, gate_up, down, per_expert_scale):
    B, L, D = x.shape
    K = weights.shape[-1]
    E, _, F2 = gate_up.shape
    F = F2 // 2
    N = B * L
    e_per_core = E // _CORES

    x_flat = x.reshape(N, D)
    w_flat = weights.reshape(N, K)
    idx_flat = expert_indices.reshape(N, K)

    out = pl.pallas_call(
        functools.partial(_moe_kernel, F=F),
        grid=(E,),
        in_specs=[
            pl.BlockSpec((N, K), lambda e: (0, 0)),
            pl.BlockSpec((N, K), lambda e: (0, 0)),
            pl.BlockSpec((N, D), lambda e: (0, 0)),
            pl.BlockSpec(memory_space=pltpu.SMEM),
            pl.BlockSpec((1, D, F2), lambda e: (e, 0, 0)),
            pl.BlockSpec((1, F, D), lambda e: (e, 0, 0)),
        ],
        out_specs=pl.BlockSpec((N, D), lambda e: (0, 0)),
        out_shape=jax.ShapeDtypeStruct((N, D), jnp.float32),
    )(idx_flat, w_flat, x_flat, per_expert_scale, gate_up, down)

    return out.reshape(B, L, D)


# speedup vs baseline: 1.2517x; 1.2213x over previous
import functools

import jax
import jax.numpy as jnp
from jax.experimental import pallas as pl
from jax.experimental.pallas import tpu as pltpu

_NBUF = 4


def _moe_kernel(idx_ref, w_ref, x_ref, scale_ref, gu_hbm, dw_hbm, out_ref,
                gu_buf, dw_buf, xb_ref, sem, *, F, E):
    lookahead = _NBUF - 1

    def start_copy(e, slot):
        pltpu.make_async_copy(gu_hbm.at[e], gu_buf.at[slot], sem.at[slot, 0]).start()
        pltpu.make_async_copy(dw_hbm.at[e], dw_buf.at[slot], sem.at[slot, 1]).start()

    for j in range(lookahead):
        start_copy(j, j)

    out_ref[...] = jnp.zeros_like(out_ref)
    xb_ref[...] = x_ref[...].astype(jnp.bfloat16)

    def body(e, _):
        slot = jax.lax.rem(e, _NBUF)

        @pl.when(e + lookahead < E)
        def _prefetch():
            start_copy(e + lookahead, jax.lax.rem(e + lookahead, _NBUF))

        pltpu.make_async_copy(gu_hbm.at[e], gu_buf.at[slot], sem.at[slot, 0]).wait()
        pltpu.make_async_copy(dw_hbm.at[e], dw_buf.at[slot], sem.at[slot, 1]).wait()

        mask = idx_ref[...] == e
        coeff = jnp.sum(jnp.where(mask, w_ref[...], 0.0), axis=1, keepdims=True)
        coeff = coeff * scale_ref[e]

        gu = gu_buf[slot].astype(jnp.bfloat16)
        h = jnp.dot(xb_ref[...], gu, preferred_element_type=jnp.float32)
        gate = h[:, :F]
        up = h[:, F:]
        act = 0.5 * gate * (1.0 + jax.lax.erf(gate * 0.7071067811865476)) * up
        y = jnp.dot(act.astype(jnp.bfloat16), dw_buf[slot].astype(jnp.bfloat16),
                    preferred_element_type=jnp.float32)
        out_ref[...] += coeff * y
        return 0

    jax.lax.fori_loop(0, E, body, 0)


def kernel(x, weights, expert_indices, gate_up, down, per_expert_scale):
    B, L, D = x.shape
    K = weights.shape[-1]
    E, _, F2 = gate_up.shape
    F = F2 // 2
    N = B * L

    x_flat = x.reshape(N, D)
    w_flat = weights.reshape(N, K)
    idx_flat = expert_indices.reshape(N, K)

    out = pl.pallas_call(
        functools.partial(_moe_kernel, F=F, E=E),
        in_specs=[
            pl.BlockSpec(memory_space=pltpu.MemorySpace.VMEM),
            pl.BlockSpec(memory_space=pltpu.MemorySpace.VMEM),
            pl.BlockSpec(memory_space=pltpu.MemorySpace.VMEM),
            pl.BlockSpec(memory_space=pltpu.MemorySpace.SMEM),
            pl.BlockSpec(memory_space=pl.ANY),
            pl.BlockSpec(memory_space=pl.ANY),
        ],
        out_specs=pl.BlockSpec(memory_space=pltpu.MemorySpace.VMEM),
        out_shape=jax.ShapeDtypeStruct((N, D), jnp.float32),
        scratch_shapes=[
            pltpu.VMEM((_NBUF, D, F2), jnp.float32),
            pltpu.VMEM((_NBUF, F, D), jnp.float32),
            pltpu.VMEM((N, D), jnp.bfloat16),
            pltpu.SemaphoreType.DMA((_NBUF, 2)),
        ],
    )(idx_flat, w_flat, x_flat, per_expert_scale, gate_up, down)

    return out.reshape(B, L, D)
